# Initial kernel scaffold; baseline (speedup 1.0000x reference)
import jax
import jax.numpy as jnp
from jax.experimental import pallas as pl


def kernel(indices, weight):
    raise NotImplementedError("write your pallas kernel here")


# speedup vs baseline: 7.3096x; 7.3096x over previous
import functools

import jax
import jax.numpy as jnp
from jax import lax
from jax.experimental import pallas as pl
from jax.experimental.pallas import tpu as pltpu
from jax.experimental.pallas import tpu_sc as plsc

_D = 128
_N = 1024 * 200
_NW = 32
_PER_W = _N // _NW
_CHUNK = 400
_NCHUNK = _PER_W // _CHUNK

_mesh = plsc.VectorSubcoreMesh(core_axis_name="c", subcore_axis_name="s")


@functools.partial(
    pl.kernel,
    mesh=_mesh,
    out_type=jax.ShapeDtypeStruct((_N, _D), jnp.float32),
    scratch_types=[
        pltpu.VMEM((_PER_W,), jnp.int32),
        pltpu.VMEM((_CHUNK, _D), jnp.float32),
        pltpu.SemaphoreType.DMA,
    ],
)
def _emb_gather(idx_hbm, tab_hbm, out_hbm, idx_v, buf, gsem):
    wid = lax.axis_index("s") * 2 + lax.axis_index("c")
    base = wid * _PER_W
    pltpu.sync_copy(idx_hbm.at[pl.ds(base, _PER_W)], idx_v)

    def body(i, carry):
        off = i * _CHUNK
        pltpu.async_copy(tab_hbm.at[idx_v.at[pl.ds(off, _CHUNK)]], buf, gsem).wait()
        pltpu.sync_copy(buf, out_hbm.at[pl.ds(base + off, _CHUNK)])
        return carry

    lax.fori_loop(0, _NCHUNK, body, 0)


def kernel(indices, weight):
    flat = indices.reshape(-1)
    out = _emb_gather(flat, weight)
    return out.reshape(indices.shape + (weight.shape[-1],))


# speedup vs baseline: 7.8746x; 1.0773x over previous
import functools

import jax
import jax.numpy as jnp
from jax import lax
from jax.experimental import pallas as pl
from jax.experimental.pallas import tpu as pltpu
from jax.experimental.pallas import tpu_sc as plsc

_D = 128
_N = 1024 * 200
_NW = 32
_PER_W = _N // _NW
_CHUNK = 800
_NCHUNK = _PER_W // _CHUNK

_mesh = plsc.VectorSubcoreMesh(core_axis_name="c", subcore_axis_name="s")


@functools.partial(
    pl.kernel,
    mesh=_mesh,
    out_type=jax.ShapeDtypeStruct((_N, _D), jnp.float32),
    scratch_types=[
        pltpu.VMEM((_PER_W,), jnp.int32),
        pltpu.VMEM((_CHUNK, _D), jnp.float32),
        pltpu.SemaphoreType.DMA,
    ],
)
def _emb_gather(idx_hbm, tab_hbm, out_hbm, idx_v, buf, gsem):
    wid = lax.axis_index("s") * 2 + lax.axis_index("c")
    base = wid * _PER_W
    pltpu.sync_copy(idx_hbm.at[pl.ds(base, _PER_W)], idx_v)

    def body(i, carry):
        off = i * _CHUNK
        pltpu.async_copy(tab_hbm.at[idx_v.at[pl.ds(off, _CHUNK)]], buf, gsem).wait()
        pltpu.sync_copy(buf, out_hbm.at[pl.ds(base + off, _CHUNK)])
        return carry

    lax.fori_loop(0, _NCHUNK, body, 0)


def kernel(indices, weight):
    flat = indices.reshape(-1)
    out = _emb_gather(flat, weight)
    return out.reshape(indices.shape + (weight.shape[-1],))
